# baseline (speedup 1.0000x reference)
import functools

import jax
import jax.numpy as jnp
from jax.experimental import pallas as pl
from jax.experimental.pallas import tpu as pltpu


def _conv_pool_kernel(x_ref, w_ref, maxp_ref, minp_ref, stats_ref,
                      xpad_ref, acc_ref, *, KH, KW, H, W):
    Cout = maxp_ref.shape[2]
    Cin = x_ref.shape[1]
    HW = H * W
    Ho2, Wo2 = H // 2, W // 2

    xpad_ref[...] = jnp.zeros_like(xpad_ref)
    xrm = jnp.transpose(x_ref[0], (1, 2, 0))
    xpad_ref[pl.ds(1, H), pl.ds(1, W), :] = xrm

    cols = []
    for kh in range(KH):
        for kw in range(KW):
            cols.append(
                xpad_ref[pl.ds(kh, H), pl.ds(kw, W), :].reshape(HW, Cin))
    patch = jnp.concatenate(cols, axis=1).astype(jnp.bfloat16)

    acc = jnp.dot(patch, w_ref[...],
                  preferred_element_type=jnp.float32)

    ssum = jnp.sum(acc, axis=0, keepdims=True)
    ssq = jnp.sum(acc * acc, axis=0, keepdims=True)
    stats_ref[...] = jnp.concatenate([ssum, ssq], axis=0).reshape(1, 2, Cout)

    acc_ref[...] = acc.reshape(H, W, Cout)
    mx = None
    mn = None
    for di in range(2):
        for dj in range(2):
            part = acc_ref[pl.ds(di, Ho2, 2), pl.ds(dj, Wo2, 2), :]
            mx = part if mx is None else jnp.maximum(mx, part)
            mn = part if mn is None else jnp.minimum(mn, part)
    maxp_ref[...] = mx.reshape(1, Ho2 * Wo2, Cout).astype(maxp_ref.dtype)
    minp_ref[...] = mn.reshape(1, Ho2 * Wo2, Cout).astype(minp_ref.dtype)


def _bn_relu_kernel(stats_ref, g_ref, be_ref, maxp_ref, minp_ref, o_ref,
                    *, count, eps):
    _, Cout, P = o_ref.shape

    ssum = jnp.sum(stats_ref[:, 0, :], axis=0, keepdims=True)
    ssq = jnp.sum(stats_ref[:, 1, :], axis=0, keepdims=True)
    mean = ssum / count
    var = ssq / count - mean * mean
    inv = jax.lax.rsqrt(var + eps)
    scale = g_ref[...] * inv
    shift = be_ref[...] - mean * scale

    sel = jnp.where(scale >= 0.0,
                    maxp_ref[0].astype(jnp.float32),
                    minp_ref[0].astype(jnp.float32))
    y = jnp.maximum(sel * scale + shift, 0.0)

    o_ref[...] = y.T.reshape(1, Cout, P)


def kernel(x, w, b, gamma, beta):
    del b
    eps = 1e-5
    N, Cin, H, W = x.shape
    Cout, _, KH, KW = w.shape
    assert H % 2 == 0 and W % 2 == 0
    Ho2, Wo2 = H // 2, W // 2
    HW = H * W
    P = Ho2 * Wo2

    wmat = jnp.transpose(w, (2, 3, 1, 0)).astype(jnp.bfloat16)
    wmat = wmat.reshape(KH * KW * Cin, Cout)
    g2 = gamma.reshape(1, Cout).astype(jnp.float32)
    be2 = beta.reshape(1, Cout).astype(jnp.float32)

    k1 = functools.partial(_conv_pool_kernel, KH=KH, KW=KW, H=H, W=W)
    flops1 = 2 * N * HW * KH * KW * Cin * Cout
    bytes1 = (4 * x.size + 2 * wmat.size
              + 2 * 2 * N * P * Cout + 4 * 2 * N * Cout)
    maxp, minp, stats = pl.pallas_call(
        k1,
        grid=(N,),
        in_specs=[
            pl.BlockSpec((1, Cin, H, W), lambda n: (n, 0, 0, 0)),
            pl.BlockSpec((KH * KW * Cin, Cout), lambda n: (0, 0)),
        ],
        out_specs=[
            pl.BlockSpec((1, P, Cout), lambda n: (n, 0, 0)),
            pl.BlockSpec((1, P, Cout), lambda n: (n, 0, 0)),
            pl.BlockSpec((1, 2, Cout), lambda n: (n, 0, 0)),
        ],
        out_shape=[
            jax.ShapeDtypeStruct((N, P, Cout), jnp.bfloat16),
            jax.ShapeDtypeStruct((N, P, Cout), jnp.bfloat16),
            jax.ShapeDtypeStruct((N, 2, Cout), jnp.float32),
        ],
        scratch_shapes=[
            pltpu.VMEM((H + 2, W + 2, Cin), jnp.float32),
            pltpu.VMEM((H, W, Cout), jnp.float32),
        ],
        compiler_params=pltpu.CompilerParams(dimension_semantics=("parallel",)),
        cost_estimate=pl.CostEstimate(flops=flops1, transcendentals=0,
                                      bytes_accessed=bytes1),
    )(x, wmat)

    count = N * HW
    k2 = functools.partial(_bn_relu_kernel, count=count, eps=eps)
    flops2 = 8 * N * P * Cout
    bytes2 = (4 * stats.size + 4 * 2 * Cout
              + 2 * 2 * N * P * Cout + 4 * N * P * Cout)
    out = pl.pallas_call(
        k2,
        grid=(N,),
        in_specs=[
            pl.BlockSpec((N, 2, Cout), lambda n: (0, 0, 0)),
            pl.BlockSpec((1, Cout), lambda n: (0, 0)),
            pl.BlockSpec((1, Cout), lambda n: (0, 0)),
            pl.BlockSpec((1, P, Cout), lambda n: (n, 0, 0)),
            pl.BlockSpec((1, P, Cout), lambda n: (n, 0, 0)),
        ],
        out_specs=pl.BlockSpec((1, Cout, P), lambda n: (n, 0, 0)),
        out_shape=jax.ShapeDtypeStruct((N, Cout, P), jnp.float32),
        compiler_params=pltpu.CompilerParams(dimension_semantics=("parallel",)),
        cost_estimate=pl.CostEstimate(flops=flops2, transcendentals=0,
                                      bytes_accessed=bytes2),
    )(stats, g2, be2, maxp, minp)

    return out.reshape(N, Cout, Ho2, Wo2)


# speedup vs baseline: 1.2891x; 1.2891x over previous
import functools

import jax
import jax.numpy as jnp
from jax.experimental import pallas as pl
from jax.experimental.pallas import tpu as pltpu


def _conv_pool_kernel(x_ref, w_ref, maxp_ref, minp_ref, stats_ref,
                      xpad_ref, acc_ref, *, KH, KW, H, W, B):
    Cout = maxp_ref.shape[2]
    Cin = x_ref.shape[1]
    HW = H * W
    Ho2, Wo2 = H // 2, W // 2

    xpad_ref[...] = jnp.zeros_like(xpad_ref)
    for e in range(B):
        xrm = x_ref[e].T
        xpad_ref[e, pl.ds(1, H), pl.ds(1, W), :] = xrm.reshape(H, W, Cin)

    cols = []
    for kh in range(KH):
        for kw in range(KW):
            cols.append(
                xpad_ref[:, pl.ds(kh, H), pl.ds(kw, W), :]
                .reshape(B * HW, Cin))
    patch = jnp.concatenate(cols, axis=1).astype(jnp.bfloat16)

    acc = jnp.dot(patch, w_ref[...],
                  preferred_element_type=jnp.float32)

    rows = []
    for e in range(B):
        blk = acc[e * HW:(e + 1) * HW, :]
        ssum = jnp.sum(blk, axis=0, keepdims=True)
        ssq = jnp.sum(blk * blk, axis=0, keepdims=True)
        rows.append(jnp.concatenate([ssum, ssq], axis=0))
    stats_ref[...] = jnp.concatenate(rows, axis=0).reshape(B, 2, Cout)

    acc_ref[...] = acc.reshape(B * H, W, Cout)
    mxs = []
    mns = []
    for e in range(B):
        mx = None
        mn = None
        for di in range(2):
            for dj in range(2):
                part = acc_ref[pl.ds(e * H + di, Ho2, 2),
                               pl.ds(dj, Wo2, 2), :]
                mx = part if mx is None else jnp.maximum(mx, part)
                mn = part if mn is None else jnp.minimum(mn, part)
        mxs.append(mx.reshape(1, Ho2 * Wo2, Cout))
        mns.append(mn.reshape(1, Ho2 * Wo2, Cout))
    maxp_ref[...] = jnp.concatenate(mxs, axis=0).astype(maxp_ref.dtype)
    minp_ref[...] = jnp.concatenate(mns, axis=0).astype(minp_ref.dtype)


def _bn_relu_kernel(stats_ref, g_ref, be_ref, maxp_ref, minp_ref, o_ref,
                    *, count, eps):
    _, Cout, P = o_ref.shape

    ssum = jnp.sum(stats_ref[:, 0, :], axis=0, keepdims=True)
    ssq = jnp.sum(stats_ref[:, 1, :], axis=0, keepdims=True)
    mean = ssum / count
    var = ssq / count - mean * mean
    inv = jax.lax.rsqrt(var + eps)
    scale = g_ref[...] * inv
    shift = be_ref[...] - mean * scale

    sel = jnp.where(scale >= 0.0,
                    maxp_ref[0].astype(jnp.float32),
                    minp_ref[0].astype(jnp.float32))
    y = jnp.maximum(sel * scale + shift, 0.0)

    o_ref[...] = y.T.reshape(1, Cout, P)


def kernel(x, w, b, gamma, beta):
    del b
    eps = 1e-5
    N, Cin, H, W = x.shape
    Cout, _, KH, KW = w.shape
    assert H % 2 == 0 and W % 2 == 0
    Ho2, Wo2 = H // 2, W // 2
    HW = H * W
    P = Ho2 * Wo2

    x3 = x.reshape(N, Cin, HW)
    wmat = jnp.transpose(w, (2, 3, 1, 0)).astype(jnp.bfloat16)
    wmat = wmat.reshape(KH * KW * Cin, Cout)
    g2 = gamma.reshape(1, Cout).astype(jnp.float32)
    be2 = beta.reshape(1, Cout).astype(jnp.float32)

    B = 2 if N % 2 == 0 else 1
    k1 = functools.partial(_conv_pool_kernel, KH=KH, KW=KW, H=H, W=W, B=B)
    flops1 = 2 * N * HW * KH * KW * Cin * Cout
    bytes1 = (4 * x3.size + 2 * wmat.size
              + 2 * 2 * N * P * Cout + 4 * 2 * N * Cout)
    maxp, minp, stats = pl.pallas_call(
        k1,
        grid=(N // B,),
        in_specs=[
            pl.BlockSpec((B, Cin, HW), lambda n: (n, 0, 0)),
            pl.BlockSpec((KH * KW * Cin, Cout), lambda n: (0, 0)),
        ],
        out_specs=[
            pl.BlockSpec((B, P, Cout), lambda n: (n, 0, 0)),
            pl.BlockSpec((B, P, Cout), lambda n: (n, 0, 0)),
            pl.BlockSpec((B, 2, Cout), lambda n: (n, 0, 0)),
        ],
        out_shape=[
            jax.ShapeDtypeStruct((N, P, Cout), jnp.bfloat16),
            jax.ShapeDtypeStruct((N, P, Cout), jnp.bfloat16),
            jax.ShapeDtypeStruct((N, 2, Cout), jnp.float32),
        ],
        scratch_shapes=[
            pltpu.VMEM((B, H + 2, W + 2, Cin), jnp.float32),
            pltpu.VMEM((B * H, W, Cout), jnp.float32),
        ],
        compiler_params=pltpu.CompilerParams(dimension_semantics=("parallel",)),
        cost_estimate=pl.CostEstimate(flops=flops1, transcendentals=0,
                                      bytes_accessed=bytes1),
    )(x3, wmat)

    count = N * HW
    k2 = functools.partial(_bn_relu_kernel, count=count, eps=eps)
    flops2 = 8 * N * P * Cout
    bytes2 = (4 * stats.size + 4 * 2 * Cout
              + 2 * 2 * N * P * Cout + 4 * N * P * Cout)
    out = pl.pallas_call(
        k2,
        grid=(N,),
        in_specs=[
            pl.BlockSpec((N, 2, Cout), lambda n: (0, 0, 0)),
            pl.BlockSpec((1, Cout), lambda n: (0, 0)),
            pl.BlockSpec((1, Cout), lambda n: (0, 0)),
            pl.BlockSpec((1, P, Cout), lambda n: (n, 0, 0)),
            pl.BlockSpec((1, P, Cout), lambda n: (n, 0, 0)),
        ],
        out_specs=pl.BlockSpec((1, Cout, P), lambda n: (n, 0, 0)),
        out_shape=jax.ShapeDtypeStruct((N, Cout, P), jnp.float32),
        compiler_params=pltpu.CompilerParams(dimension_semantics=("parallel",)),
        cost_estimate=pl.CostEstimate(flops=flops2, transcendentals=0,
                                      bytes_accessed=bytes2),
    )(stats, g2, be2, maxp, minp)

    return out.reshape(N, Cout, Ho2, Wo2)


# speedup vs baseline: 1.4634x; 1.1352x over previous
import functools

import jax
import jax.numpy as jnp
from jax.experimental import pallas as pl
from jax.experimental.pallas import tpu as pltpu


def _conv_pool_kernel(xp_ref, w_ref, maxp_ref, minp_ref, stats_ref,
                      acc_ref, *, KH, KW, B):
    Cout = maxp_ref.shape[2]
    Cin = xp_ref.shape[3]
    Ho = xp_ref.shape[1] - (KH - 1)
    Wo = xp_ref.shape[2] - (KW - 1)
    Ho2, Wo2 = Ho // 2, Wo // 2
    rows = Ho * Wo

    cols = []
    for kh in range(KH):
        for kw in range(KW):
            cols.append(
                xp_ref[:, pl.ds(kh, Ho), pl.ds(kw, Wo), :]
                .reshape(B * rows, Cin))
    patch = jnp.concatenate(cols, axis=1).astype(jnp.bfloat16)

    acc = jnp.dot(patch, w_ref[...],
                  preferred_element_type=jnp.float32)

    srows = []
    for e in range(B):
        blk = acc[e * rows:(e + 1) * rows, :]
        ssum = jnp.sum(blk, axis=0, keepdims=True)
        ssq = jnp.sum(blk * blk, axis=0, keepdims=True)
        srows.append(jnp.concatenate([ssum, ssq], axis=0))
    stats_ref[...] = jnp.concatenate(srows, axis=0).reshape(B, 2, Cout)

    acc_ref[...] = acc.reshape(B * Ho, Wo, Cout)
    mxs = []
    mns = []
    for e in range(B):
        mx = None
        mn = None
        for di in range(2):
            for dj in range(2):
                part = acc_ref[pl.ds(e * Ho + di, Ho2, 2),
                               pl.ds(dj, Wo2, 2), :]
                mx = part if mx is None else jnp.maximum(mx, part)
                mn = part if mn is None else jnp.minimum(mn, part)
        mxs.append(mx.reshape(1, Ho2 * Wo2, Cout))
        mns.append(mn.reshape(1, Ho2 * Wo2, Cout))
    maxp_ref[...] = jnp.concatenate(mxs, axis=0).astype(maxp_ref.dtype)
    minp_ref[...] = jnp.concatenate(mns, axis=0).astype(minp_ref.dtype)


def _bn_relu_kernel(stats_ref, g_ref, be_ref, maxp_ref, minp_ref, o_ref,
                    *, count, eps):
    _, Cout, P = o_ref.shape

    ssum = jnp.sum(stats_ref[:, 0, :], axis=0, keepdims=True)
    ssq = jnp.sum(stats_ref[:, 1, :], axis=0, keepdims=True)
    mean = ssum / count
    var = ssq / count - mean * mean
    inv = jax.lax.rsqrt(var + eps)
    scale = g_ref[...] * inv
    shift = be_ref[...] - mean * scale

    sel = jnp.where(scale >= 0.0,
                    maxp_ref[0].astype(jnp.float32),
                    minp_ref[0].astype(jnp.float32))
    y = jnp.maximum(sel * scale + shift, 0.0)

    o_ref[...] = y.T.reshape(1, Cout, P)


def kernel(x, w, b, gamma, beta):
    del b
    stride, padding, eps = 1, 1, 1e-5
    N, Cin, H, W = x.shape
    Cout, _, KH, KW = w.shape
    Ho = (H + 2 * padding - KH) // stride + 1
    Wo = (W + 2 * padding - KW) // stride + 1
    assert Ho % 2 == 0 and Wo % 2 == 0
    Ho2, Wo2 = Ho // 2, Wo // 2
    P = Ho2 * Wo2

    xn = jnp.transpose(x, (0, 2, 3, 1)).astype(jnp.float32)
    xp = jnp.pad(xn, ((0, 0), (padding, padding), (padding, padding), (0, 0)))
    wmat = jnp.transpose(w, (2, 3, 1, 0)).astype(jnp.bfloat16)
    wmat = wmat.reshape(KH * KW * Cin, Cout)
    g2 = gamma.reshape(1, Cout).astype(jnp.float32)
    be2 = beta.reshape(1, Cout).astype(jnp.float32)
    Hp, Wp = xp.shape[1], xp.shape[2]

    B = 2 if N % 2 == 0 else 1
    k1 = functools.partial(_conv_pool_kernel, KH=KH, KW=KW, B=B)
    flops1 = 2 * N * Ho * Wo * KH * KW * Cin * Cout
    bytes1 = (4 * xp.size + 2 * wmat.size
              + 2 * 2 * N * P * Cout + 4 * 2 * N * Cout)
    maxp, minp, stats = pl.pallas_call(
        k1,
        grid=(N // B,),
        in_specs=[
            pl.BlockSpec((B, Hp, Wp, Cin), lambda n: (n, 0, 0, 0)),
            pl.BlockSpec((KH * KW * Cin, Cout), lambda n: (0, 0)),
        ],
        out_specs=[
            pl.BlockSpec((B, P, Cout), lambda n: (n, 0, 0)),
            pl.BlockSpec((B, P, Cout), lambda n: (n, 0, 0)),
            pl.BlockSpec((B, 2, Cout), lambda n: (n, 0, 0)),
        ],
        out_shape=[
            jax.ShapeDtypeStruct((N, P, Cout), jnp.bfloat16),
            jax.ShapeDtypeStruct((N, P, Cout), jnp.bfloat16),
            jax.ShapeDtypeStruct((N, 2, Cout), jnp.float32),
        ],
        scratch_shapes=[
            pltpu.VMEM((B * Ho, Wo, Cout), jnp.float32),
        ],
        compiler_params=pltpu.CompilerParams(dimension_semantics=("parallel",)),
        cost_estimate=pl.CostEstimate(flops=flops1, transcendentals=0,
                                      bytes_accessed=bytes1),
    )(xp, wmat)

    count = N * Ho * Wo
    k2 = functools.partial(_bn_relu_kernel, count=count, eps=eps)
    flops2 = 8 * N * P * Cout
    bytes2 = (4 * stats.size + 4 * 2 * Cout
              + 2 * 2 * N * P * Cout + 4 * N * P * Cout)
    out = pl.pallas_call(
        k2,
        grid=(N,),
        in_specs=[
            pl.BlockSpec((N, 2, Cout), lambda n: (0, 0, 0)),
            pl.BlockSpec((1, Cout), lambda n: (0, 0)),
            pl.BlockSpec((1, Cout), lambda n: (0, 0)),
            pl.BlockSpec((1, P, Cout), lambda n: (n, 0, 0)),
            pl.BlockSpec((1, P, Cout), lambda n: (n, 0, 0)),
        ],
        out_specs=pl.BlockSpec((1, Cout, P), lambda n: (n, 0, 0)),
        out_shape=jax.ShapeDtypeStruct((N, Cout, P), jnp.float32),
        compiler_params=pltpu.CompilerParams(dimension_semantics=("parallel",)),
        cost_estimate=pl.CostEstimate(flops=flops2, transcendentals=0,
                                      bytes_accessed=bytes2),
    )(stats, g2, be2, maxp, minp)

    return out.reshape(N, Cout, Ho2, Wo2)


# speedup vs baseline: 1.6234x; 1.1093x over previous
import functools

import jax
import jax.numpy as jnp
from jax.experimental import pallas as pl
from jax.experimental.pallas import tpu as pltpu


def _conv_pool_kernel(xp_ref, w_ref, maxp_ref, minp_ref, stats_ref,
                      acc_ref, *, KH, KW, B):
    Cout = maxp_ref.shape[2]
    Cin = xp_ref.shape[3]
    Ho = xp_ref.shape[1] - (KH - 1)
    Wo = xp_ref.shape[2] - (KW - 1)
    Ho2, Wo2 = Ho // 2, Wo // 2
    rows = Ho * Wo

    cols = []
    for kh in range(KH):
        for kw in range(KW):
            cols.append(
                xp_ref[:, pl.ds(kh, Ho), pl.ds(kw, Wo), :]
                .reshape(B * rows, Cin))
    patch = jnp.concatenate(cols, axis=1).astype(jnp.bfloat16)

    acc = jnp.dot(patch, w_ref[...],
                  preferred_element_type=jnp.float32)

    srows = []
    for e in range(B):
        blk = acc[e * rows:(e + 1) * rows, :]
        ssum = jnp.sum(blk, axis=0, keepdims=True)
        ssq = jnp.sum(blk * blk, axis=0, keepdims=True)
        srows.append(jnp.concatenate([ssum, ssq], axis=0))
    stats_ref[...] = jnp.concatenate(srows, axis=0).reshape(B, 2, Cout)

    acc_ref[...] = acc.reshape(B * Ho, Wo, Cout)
    mxs = []
    mns = []
    for e in range(B):
        mx = None
        mn = None
        for di in range(2):
            for dj in range(2):
                part = acc_ref[pl.ds(e * Ho + di, Ho2, 2),
                               pl.ds(dj, Wo2, 2), :]
                mx = part if mx is None else jnp.maximum(mx, part)
                mn = part if mn is None else jnp.minimum(mn, part)
        mxs.append(mx.reshape(1, Ho2 * Wo2, Cout))
        mns.append(mn.reshape(1, Ho2 * Wo2, Cout))
    maxp_ref[...] = jnp.concatenate(mxs, axis=0).astype(maxp_ref.dtype)
    minp_ref[...] = jnp.concatenate(mns, axis=0).astype(minp_ref.dtype)


def _bn_relu_kernel(stats_ref, g_ref, be_ref, maxp_ref, minp_ref, o_ref,
                    *, count, eps, B2):
    _, Cout, P = o_ref.shape

    ssum = jnp.sum(stats_ref[:, 0, :], axis=0, keepdims=True)
    ssq = jnp.sum(stats_ref[:, 1, :], axis=0, keepdims=True)
    mean = ssum / count
    var = ssq / count - mean * mean
    inv = jax.lax.rsqrt(var + eps)
    scale = g_ref[...] * inv
    shift = be_ref[...] - mean * scale

    for e in range(B2):
        sel = jnp.where(scale >= 0.0,
                        maxp_ref[e].astype(jnp.float32),
                        minp_ref[e].astype(jnp.float32))
        y = jnp.maximum(sel * scale + shift, 0.0)
        o_ref[e] = y.T


def kernel(x, w, b, gamma, beta):
    del b
    stride, padding, eps = 1, 1, 1e-5
    N, Cin, H, W = x.shape
    Cout, _, KH, KW = w.shape
    Ho = (H + 2 * padding - KH) // stride + 1
    Wo = (W + 2 * padding - KW) // stride + 1
    assert Ho % 2 == 0 and Wo % 2 == 0
    Ho2, Wo2 = Ho // 2, Wo // 2
    P = Ho2 * Wo2

    xn = jnp.transpose(x, (0, 2, 3, 1)).astype(jnp.float32)
    xp = jnp.pad(xn, ((0, 0), (padding, padding), (padding, padding), (0, 0)))
    wmat = jnp.transpose(w, (2, 3, 1, 0)).astype(jnp.bfloat16)
    wmat = wmat.reshape(KH * KW * Cin, Cout)
    g2 = gamma.reshape(1, Cout).astype(jnp.float32)
    be2 = beta.reshape(1, Cout).astype(jnp.float32)
    Hp, Wp = xp.shape[1], xp.shape[2]

    B = 2 if N % 2 == 0 else 1
    k1 = functools.partial(_conv_pool_kernel, KH=KH, KW=KW, B=B)
    flops1 = 2 * N * Ho * Wo * KH * KW * Cin * Cout
    bytes1 = (4 * xp.size + 2 * wmat.size
              + 2 * 2 * N * P * Cout + 4 * 2 * N * Cout)
    maxp, minp, stats = pl.pallas_call(
        k1,
        grid=(N // B,),
        in_specs=[
            pl.BlockSpec((B, Hp, Wp, Cin), lambda n: (n, 0, 0, 0)),
            pl.BlockSpec((KH * KW * Cin, Cout), lambda n: (0, 0)),
        ],
        out_specs=[
            pl.BlockSpec((B, P, Cout), lambda n: (n, 0, 0)),
            pl.BlockSpec((B, P, Cout), lambda n: (n, 0, 0)),
            pl.BlockSpec((B, 2, Cout), lambda n: (n, 0, 0)),
        ],
        out_shape=[
            jax.ShapeDtypeStruct((N, P, Cout), jnp.bfloat16),
            jax.ShapeDtypeStruct((N, P, Cout), jnp.bfloat16),
            jax.ShapeDtypeStruct((N, 2, Cout), jnp.float32),
        ],
        scratch_shapes=[
            pltpu.VMEM((B * Ho, Wo, Cout), jnp.float32),
        ],
        compiler_params=pltpu.CompilerParams(dimension_semantics=("parallel",)),
        cost_estimate=pl.CostEstimate(flops=flops1, transcendentals=0,
                                      bytes_accessed=bytes1),
    )(xp, wmat)

    count = N * Ho * Wo
    B2 = 8 if N % 8 == 0 else 1
    k2 = functools.partial(_bn_relu_kernel, count=count, eps=eps, B2=B2)
    flops2 = 8 * N * P * Cout
    bytes2 = (4 * stats.size + 4 * 2 * Cout
              + 2 * 2 * N * P * Cout + 4 * N * P * Cout)
    out = pl.pallas_call(
        k2,
        grid=(N // B2,),
        in_specs=[
            pl.BlockSpec((N, 2, Cout), lambda n: (0, 0, 0)),
            pl.BlockSpec((1, Cout), lambda n: (0, 0)),
            pl.BlockSpec((1, Cout), lambda n: (0, 0)),
            pl.BlockSpec((B2, P, Cout), lambda n: (n, 0, 0)),
            pl.BlockSpec((B2, P, Cout), lambda n: (n, 0, 0)),
        ],
        out_specs=pl.BlockSpec((B2, Cout, P), lambda n: (n, 0, 0)),
        out_shape=jax.ShapeDtypeStruct((N, Cout, P), jnp.float32),
        compiler_params=pltpu.CompilerParams(dimension_semantics=("parallel",)),
        cost_estimate=pl.CostEstimate(flops=flops2, transcendentals=0,
                                      bytes_accessed=bytes2),
    )(stats, g2, be2, maxp, minp)

    return out.reshape(N, Cout, Ho2, Wo2)
